# baseline (speedup 1.0000x reference)
import functools
import jax
import jax.numpy as jnp
from jax.experimental import pallas as pl
from jax.experimental.pallas import tpu as pltpu

N = 19683
STATE = 64
K = 26
GATE_H = 32
MSG_H = 32
INTEGRATION_STEPS = 3
BN = 729


def _moe_block(ns_ref, cs_ref,
               wg1_ref, bg1_ref, wg2_ref, bg2_ref,
               wl_ref, bl_ref, wm_ref, bm_ref,
               wu_ref, bu_ref, wc_ref, bc_ref,
               out_ref, gate_ref):
    f32 = jnp.float32
    x = ns_ref[0]
    acc = x[:, 0:128]
    for i in range(1, 13):
        acc = acc + x[:, 128 * i:128 * (i + 1)]
    nmean = (acc[:, 0:64] + acc[:, 64:128]) * f32(1.0 / K)
    cs = cs_ref[0]
    combined = jnp.concatenate([cs, nmean], axis=-1)

    dot = functools.partial(jnp.dot, preferred_element_type=f32)

    gate_h = jnp.tanh(dot(combined, wg1_ref[...]) + bg1_ref[...])
    logits = dot(gate_h, wg2_ref[...]) + bg2_ref[...]
    m = jnp.max(logits, axis=-1, keepdims=True)
    e = jnp.exp(logits - m)
    gate_w = e / jnp.sum(e, axis=-1, keepdims=True)

    local_out = jnp.tanh(dot(combined, wl_ref[...]) + bl_ref[...])

    msg = jnp.tanh(dot(combined, wm_ref[...]) + bm_ref[...])
    func_out = jnp.tanh(dot(cs, wu_ref[:STATE, :]) +
                        dot(msg, wu_ref[STATE:, :]) + bu_ref[...])

    cnf_base = dot(nmean, wc_ref[STATE:, :]) + bc_ref[...]
    dt = f32(1.0 / INTEGRATION_STEPS)
    s = cs
    for _ in range(INTEGRATION_STEPS):
        ds = jnp.tanh(dot(s, wc_ref[:STATE, :]) + cnf_base)
        s = s + dt * ds

    out_ref[0] = (gate_w[:, 0:1] * local_out
                  + gate_w[:, 1:2] * func_out
                  + gate_w[:, 2:3] * s)
    gate_ref[0] = gate_w


@jax.jit
def kernel(current_state, neighbor_states,
           W_gate1, b_gate1, W_gate2, b_gate2,
           W_local, b_local,
           W_msg, b_msg, W_upd, b_upd,
           W_cnf, b_cnf):
    nblocks = N // BN
    ns_flat = neighbor_states.reshape(nblocks, BN, K * STATE)
    cs3 = current_state.reshape(nblocks, BN, STATE)
    grid = (nblocks,)

    def rows(i):
        return (i, 0, 0)

    def whole(i):
        return (0, 0)

    full = lambda shape: pl.BlockSpec(shape, whole)
    out_state, gate_w = pl.pallas_call(
        _moe_block,
        grid=grid,
        in_specs=[
            pl.BlockSpec((1, BN, K * STATE), rows),
            pl.BlockSpec((1, BN, STATE), rows),
            full((2 * STATE, GATE_H)), full((1, GATE_H)),
            full((GATE_H, 3)), full((1, 3)),
            full((2 * STATE, STATE)), full((1, STATE)),
            full((2 * STATE, MSG_H)), full((1, MSG_H)),
            full((STATE + MSG_H, STATE)), full((1, STATE)),
            full((2 * STATE, STATE)), full((1, STATE)),
        ],
        out_specs=[
            pl.BlockSpec((1, BN, STATE), rows),
            pl.BlockSpec((1, BN, 3), rows),
        ],
        out_shape=[
            jax.ShapeDtypeStruct((nblocks, BN, STATE), jnp.float32),
            jax.ShapeDtypeStruct((nblocks, BN, 3), jnp.float32),
        ],
        compiler_params=pltpu.CompilerParams(
            dimension_semantics=("arbitrary",),
        ),
    )(ns_flat, cs3,
      W_gate1, b_gate1.reshape(1, -1), W_gate2, b_gate2.reshape(1, -1),
      W_local, b_local.reshape(1, -1), W_msg, b_msg.reshape(1, -1),
      W_upd, b_upd.reshape(1, -1), W_cnf, b_cnf.reshape(1, -1))
    return out_state.reshape(N, STATE), gate_w.reshape(N, 3)
